# Initial kernel scaffold; baseline (speedup 1.0000x reference)
import jax
import jax.numpy as jnp
from jax.experimental import pallas as pl


def kernel(x, edge_index, node_idx, labels, p_hol, p_shared, combine_weight, p_balance, W1, W2, alpha):
    raise NotImplementedError("write your pallas kernel here")


# speedup vs baseline: 2.1328x; 2.1328x over previous
import functools
import jax
import jax.numpy as jnp
from jax import lax
from jax.experimental import pallas as pl
from jax.experimental.pallas import tpu as pltpu

TEMP = 0.2
EPS = 1e-8
_INTERPRET = False

NEG = -3.0e38


def _elu_prompt_body(x_ref, pc_ref, o_ref):
    t = x_ref[...] * pc_ref[...]
    o_ref[...] = jnp.where(t > 0, t, jnp.exp(jnp.minimum(t, 0.0)) - 1.0)


def _fused_prompt(x, pc):
    n, d = x.shape
    return pl.pallas_call(
        _elu_prompt_body,
        out_shape=jax.ShapeDtypeStruct((n, d), jnp.float32),
        interpret=_INTERPRET,
    )(x, pc.reshape(1, d))


def _hn_body(fa_ref, agg_ref, dis2_ref, pb_ref, hn_ref):
    fa = fa_ref[...]
    agg = agg_ref[...] + dis2_ref[...] * fa
    h = jnp.concatenate([fa, agg], axis=1) * pb_ref[...]
    nrm = jnp.sqrt(jnp.sum(h * h, axis=1, keepdims=True))
    hn_ref[...] = h / (nrm + EPS)


def _hn_kernel(fea_al, agg, dis2, p_bal):
    n, d = fea_al.shape
    return pl.pallas_call(
        _hn_body,
        out_shape=jax.ShapeDtypeStruct((n, 2 * d), jnp.float32),
        interpret=_INTERPRET,
    )(fea_al, agg, dis2.reshape(n, 1), p_bal.reshape(1, 2 * d))


def _topk_body(nvalid, k, kp, blk_ref, hnT_ref, val_ref, idx_ref):
    blk = blk_ref[...]
    sims = jnp.dot(blk, hnT_ref[...], preferred_element_type=jnp.float32)
    r, npad = sims.shape
    col = lax.broadcasted_iota(jnp.int32, (r, npad), 1)
    sims = jnp.where(col >= nvalid, NEG, sims)
    kcol = lax.broadcasted_iota(jnp.int32, (r, kp), 1)

    def body(j, carry):
        s, vals, idxs = carry
        m = jnp.max(s, axis=1, keepdims=True)
        am = jnp.min(jnp.where(s == m, col, npad), axis=1, keepdims=True)
        vals = jnp.where(kcol == j, m, vals)
        idxs = jnp.where(kcol == j, am, idxs)
        s = jnp.where(col == am, NEG, s)
        return s, vals, idxs

    _, vals, idxs = lax.fori_loop(
        0, k,
        body,
        (sims, jnp.zeros((r, kp), jnp.float32), jnp.zeros((r, kp), jnp.int32)),
    )
    val_ref[...] = vals
    idx_ref[...] = idxs


def _knn_topk(hn_pad, nvalid, k, kp, rblk):
    npad, d2 = hn_pad.shape
    nb = npad // rblk
    hnT = hn_pad.T
    body = functools.partial(_topk_body, nvalid, k, kp)
    return pl.pallas_call(
        body,
        grid=(nb,),
        in_specs=[
            pl.BlockSpec((rblk, d2), lambda i: (i, 0)),
            pl.BlockSpec((d2, npad), lambda i: (0, 0)),
        ],
        out_specs=[
            pl.BlockSpec((rblk, kp), lambda i: (i, 0)),
            pl.BlockSpec((rblk, kp), lambda i: (i, 0)),
        ],
        out_shape=[
            jax.ShapeDtypeStruct((npad, kp), jnp.float32),
            jax.ShapeDtypeStruct((npad, kp), jnp.int32),
        ],
        interpret=_INTERPRET,
    )(hn_pad, hnT)


def _mm_body(a_ref, b_ref, o_ref):
    o_ref[...] = jnp.dot(a_ref[...], b_ref[...], preferred_element_type=jnp.float32)


def _matmul(a, b):
    m, k = a.shape
    k2, n = b.shape
    return pl.pallas_call(
        _mm_body,
        out_shape=jax.ShapeDtypeStruct((m, n), jnp.float32),
        interpret=_INTERPRET,
    )(a, b)


def _head1_body(ohT_ref, sel_ref, an_ref, bn_ref):
    sel = sel_ref[...]
    ohT = ohT_ref[...]
    sums = jnp.dot(ohT, sel, preferred_element_type=jnp.float32)
    cnts = jnp.sum(ohT, axis=1, keepdims=True)
    proto = sums / jnp.maximum(cnts, 1.0)
    bn_ref[...] = proto / (jnp.sqrt(jnp.sum(proto * proto, axis=1, keepdims=True)) + EPS)
    an_ref[...] = sel / (jnp.sqrt(jnp.sum(sel * sel, axis=1, keepdims=True)) + EPS)


def _head1(onehotT, sel):
    c, nsel = onehotT.shape
    _, h = sel.shape
    return pl.pallas_call(
        _head1_body,
        out_shape=[
            jax.ShapeDtypeStruct((nsel, h), jnp.float32),
            jax.ShapeDtypeStruct((c, h), jnp.float32),
        ],
        interpret=_INTERPRET,
    )(onehotT, sel)


def _head2_body(an_ref, bnT_ref, o_ref):
    o_ref[...] = jnp.dot(an_ref[...], bnT_ref[...],
                         preferred_element_type=jnp.float32) * (1.0 / TEMP)


def _head2(an, bnT):
    nsel, h = an.shape
    _, c = bnT.shape
    return pl.pallas_call(
        _head2_body,
        out_shape=jax.ShapeDtypeStruct((nsel, c), jnp.float32),
        interpret=_INTERPRET,
    )(an, bnT)


def kernel(x, edge_index, node_idx, labels, p_hol, p_shared, combine_weight,
           p_balance, W1, W2, alpha):
    n, d = x.shape
    kk = 33
    kp = 64
    rblk = 256
    npad = ((n + rblk - 1) // rblk) * rblk
    c = 64
    src, dst = edge_index[0], edge_index[1]
    e = src.shape[0]

    pc = combine_weight[0, 0] * p_hol + combine_weight[0, 1] * p_shared
    fea_al = _fused_prompt(x, pc)

    deg = jnp.ones((n,), jnp.float32).at[dst].add(jnp.ones((e,), jnp.float32))
    dis = deg ** -0.5
    dis2 = dis * dis
    w_e = dis[src] * dis[dst]

    agg = jnp.zeros((n, d), jnp.float32).at[dst].add(w_e[:, None] * fea_al[src])

    hn = _hn_kernel(fea_al, agg, dis2, p_balance)
    hn_pad = jnp.pad(hn, ((0, npad - n), (0, 0)))

    vals_p, idxs_p = _knn_topk(hn_pad, n, kk, kp, rblk)
    val = vals_p[:n, :kk]
    idx = idxs_p[:n, :kk]

    g = idx[idx.reshape(-1)].reshape(n, kk, kk)
    rev = jnp.any(g == jnp.arange(n, dtype=idx.dtype)[:, None, None], axis=-1)
    v = jax.nn.relu(jnp.where(rev, 0.5, 1.0) * val)

    aw_e = alpha * w_e
    a_self = (alpha * dis2)[:, None]
    bv = (1.0 - alpha) * v
    idx_flat = idx.reshape(-1)
    bv_flat = bv.reshape(-1)

    def prop(f):
        o = jnp.zeros((n, f.shape[1]), jnp.float32).at[dst].add(aw_e[:, None] * f[src])
        o = o + a_self * f
        o = o + jnp.einsum('ij,ijk->ik', bv, f[idx])
        o = o.at[idx_flat].add(bv_flat[:, None] * jnp.repeat(f, kk, axis=0))
        return o

    z1 = _matmul(fea_al, W1)
    h1 = jax.nn.relu(prop(z1))
    z2 = _matmul(h1, W2)
    out = prop(z2)

    sel = out[node_idx]
    onehotT = (labels[None, :] == jnp.arange(c, dtype=labels.dtype)[:, None]
               ).astype(jnp.float32)
    an, bn = _head1(onehotT, sel)
    return _head2(an, bn.T)


# speedup vs baseline: 2.7308x; 1.2804x over previous
import functools
import jax
import jax.numpy as jnp
from jax import lax
from jax.experimental import pallas as pl
from jax.experimental.pallas import tpu as pltpu
from jax.experimental.pallas import tpu_sc as plsc

TEMP = 0.2
EPS = 1e-8
_INTERPRET = False

NEG = -3.0e38


_SC_B = 128


def _edge_scatter_body(npad, d, ept, b, feat_hbm, src_hbm, dst_hbm, w_hbm,
                       out_hbm, src_v, dst_v, w_v, rows_v, acc_sh, sem):
    cid = lax.axis_index("c")
    sid = lax.axis_index("s")
    wid = sid * 2 + cid
    nchunks = ept // b
    rows_per_tile = npad // 16
    cd = d // 16

    zero = jnp.zeros((16,), jnp.float32)

    def zbuf(r, carry):
        for c in range(cd):
            rows_v[r, pl.ds(c * 16, 16)] = zero
        return carry

    lax.fori_loop(0, b, zbuf, 0)

    def zacc(ci, carry):
        pltpu.sync_copy(rows_v, acc_sh.at[pl.ds(sid * rows_per_tile + ci * b, b)])
        return carry

    lax.fori_loop(0, rows_per_tile // b, zacc, 0)
    plsc.subcore_barrier()

    def chunk(ci, carry):
        off = wid * ept + ci * b
        pltpu.sync_copy(src_hbm.at[pl.ds(off, b)], src_v)
        pltpu.sync_copy(dst_hbm.at[pl.ds(off, b)], dst_v)
        pltpu.sync_copy(w_hbm.at[pl.ds(off, b)], w_v)
        pltpu.async_copy(feat_hbm.at[src_v], rows_v, sem).wait()

        def scale(g, c2):
            wch = w_v[pl.ds(g * 16, 16)]
            for i in range(16):
                e = g * 16 + i
                wv = wch[i]
                for c in range(cd):
                    rows_v[e, pl.ds(c * 16, 16)] = rows_v[e, pl.ds(c * 16, 16)] * wv
            return c2

        lax.fori_loop(0, b // 16, scale, 0)
        pltpu.sync_copy(rows_v, acc_sh.at[dst_v], add=True)
        return carry

    lax.fori_loop(0, nchunks, chunk, 0)
    plsc.subcore_barrier()

    def cout(ci, carry):
        r0 = sid * rows_per_tile + ci * b
        pltpu.sync_copy(acc_sh.at[pl.ds(r0, b)], rows_v)
        pltpu.sync_copy(rows_v, out_hbm.at[cid, pl.ds(r0, b)])
        return carry

    lax.fori_loop(0, rows_per_tile // b, cout, 0)


def _edge_scatter(feat_pad, esrc, edst, ew):
    npad, d = feat_pad.shape
    e_tot = esrc.shape[0]
    b = _SC_B
    ept = ((e_tot + 32 * b - 1) // (32 * b)) * b
    e_pad = 32 * ept
    pad = e_pad - e_tot
    esrc = jnp.pad(esrc, (0, pad))
    edst = jnp.pad(edst, (0, pad), constant_values=npad - 1)
    ew = jnp.pad(ew, (0, pad))
    mesh = plsc.VectorSubcoreMesh(core_axis_name="c", subcore_axis_name="s")
    body = functools.partial(_edge_scatter_body, npad, d, ept, b)
    f = pl.kernel(
        body,
        out_type=jax.ShapeDtypeStruct((2, npad, d), jnp.float32),
        mesh=mesh,
        scratch_types=[
            pltpu.VMEM((b,), jnp.int32),
            pltpu.VMEM((b,), jnp.int32),
            pltpu.VMEM((b,), jnp.float32),
            pltpu.VMEM((b, d), jnp.float32),
            pltpu.VMEM_SHARED((npad, d), jnp.float32),
            pltpu.SemaphoreType.DMA,
        ],
        interpret=_INTERPRET,
    )
    return f(feat_pad, esrc, edst, ew)


def _elu_prompt_body(x_ref, pc_ref, o_ref):
    t = x_ref[...] * pc_ref[...]
    o_ref[...] = jnp.where(t > 0, t, jnp.exp(jnp.minimum(t, 0.0)) - 1.0)


def _fused_prompt(x, pc):
    n, d = x.shape
    return pl.pallas_call(
        _elu_prompt_body,
        out_shape=jax.ShapeDtypeStruct((n, d), jnp.float32),
        interpret=_INTERPRET,
    )(x, pc.reshape(1, d))


def _hn_body(fa_ref, p0_ref, p1_ref, dis2_ref, pb_ref, hn_ref):
    fa = fa_ref[...]
    agg = p0_ref[...] + p1_ref[...] + dis2_ref[...] * fa
    h = jnp.concatenate([fa, agg], axis=1) * pb_ref[...]
    nrm = jnp.sqrt(jnp.sum(h * h, axis=1, keepdims=True))
    hn_ref[...] = h / (nrm + EPS)


def _hn_kernel(fea_al, aggp, dis2, p_bal):
    n, d = fea_al.shape
    return pl.pallas_call(
        _hn_body,
        out_shape=jax.ShapeDtypeStruct((n, 2 * d), jnp.float32),
        interpret=_INTERPRET,
    )(fea_al, aggp[0], aggp[1], dis2.reshape(n, 1), p_bal.reshape(1, 2 * d))


def _combine_mm_body(p0_ref, p1_ref, z_ref, a2_ref, w_ref, o_ref):
    h = jax.nn.relu(p0_ref[...] + p1_ref[...] + a2_ref[...] * z_ref[...])
    o_ref[...] = jnp.dot(h, w_ref[...], preferred_element_type=jnp.float32)


def _combine_mm(prop_p, z, a2, w):
    n, d = z.shape
    return pl.pallas_call(
        _combine_mm_body,
        out_shape=jax.ShapeDtypeStruct((n, w.shape[1]), jnp.float32),
        interpret=_INTERPRET,
    )(prop_p[0], prop_p[1], z, a2.reshape(n, 1), w)


def _combine_body(p0_ref, p1_ref, z_ref, a2_ref, o_ref):
    o_ref[...] = p0_ref[...] + p1_ref[...] + a2_ref[...] * z_ref[...]


def _combine(prop_p, z, a2):
    n, d = z.shape
    return pl.pallas_call(
        _combine_body,
        out_shape=jax.ShapeDtypeStruct((n, d), jnp.float32),
        interpret=_INTERPRET,
    )(prop_p[0], prop_p[1], z, a2.reshape(n, 1))


def _topk_body(nvalid, k, kp, blk_ref, hnT_ref, val_ref, idx_ref):
    blk = blk_ref[...]
    sims = jnp.dot(blk, hnT_ref[...], preferred_element_type=jnp.float32)
    r, npad = sims.shape
    col = lax.broadcasted_iota(jnp.int32, (r, npad), 1)
    sims = jnp.where(col >= nvalid, NEG, sims)
    kcol = lax.broadcasted_iota(jnp.int32, (r, kp), 1)

    def body(j, carry):
        s, vals, idxs = carry
        m = jnp.max(s, axis=1, keepdims=True)
        am = jnp.min(jnp.where(s == m, col, npad), axis=1, keepdims=True)
        vals = jnp.where(kcol == j, m, vals)
        idxs = jnp.where(kcol == j, am, idxs)
        s = jnp.where(col == am, NEG, s)
        return s, vals, idxs

    _, vals, idxs = lax.fori_loop(
        0, k,
        body,
        (sims, jnp.zeros((r, kp), jnp.float32), jnp.zeros((r, kp), jnp.int32)),
    )
    val_ref[...] = vals
    idx_ref[...] = idxs


def _knn_topk(hn_pad, nvalid, k, kp, rblk):
    npad, d2 = hn_pad.shape
    nb = npad // rblk
    hnT = hn_pad.T
    body = functools.partial(_topk_body, nvalid, k, kp)
    return pl.pallas_call(
        body,
        grid=(nb,),
        in_specs=[
            pl.BlockSpec((rblk, d2), lambda i: (i, 0)),
            pl.BlockSpec((d2, npad), lambda i: (0, 0)),
        ],
        out_specs=[
            pl.BlockSpec((rblk, kp), lambda i: (i, 0)),
            pl.BlockSpec((rblk, kp), lambda i: (i, 0)),
        ],
        out_shape=[
            jax.ShapeDtypeStruct((npad, kp), jnp.float32),
            jax.ShapeDtypeStruct((npad, kp), jnp.int32),
        ],
        interpret=_INTERPRET,
    )(hn_pad, hnT)


def _mm_body(a_ref, b_ref, o_ref):
    o_ref[...] = jnp.dot(a_ref[...], b_ref[...], preferred_element_type=jnp.float32)


def _matmul(a, b):
    m, k = a.shape
    k2, n = b.shape
    return pl.pallas_call(
        _mm_body,
        out_shape=jax.ShapeDtypeStruct((m, n), jnp.float32),
        interpret=_INTERPRET,
    )(a, b)


def _head1_body(ohT_ref, sel_ref, an_ref, bn_ref):
    sel = sel_ref[...]
    ohT = ohT_ref[...]
    sums = jnp.dot(ohT, sel, preferred_element_type=jnp.float32)
    cnts = jnp.sum(ohT, axis=1, keepdims=True)
    proto = sums / jnp.maximum(cnts, 1.0)
    bn_ref[...] = proto / (jnp.sqrt(jnp.sum(proto * proto, axis=1, keepdims=True)) + EPS)
    an_ref[...] = sel / (jnp.sqrt(jnp.sum(sel * sel, axis=1, keepdims=True)) + EPS)


def _head1(onehotT, sel):
    c, nsel = onehotT.shape
    _, h = sel.shape
    return pl.pallas_call(
        _head1_body,
        out_shape=[
            jax.ShapeDtypeStruct((nsel, h), jnp.float32),
            jax.ShapeDtypeStruct((c, h), jnp.float32),
        ],
        interpret=_INTERPRET,
    )(onehotT, sel)


def _head2_body(an_ref, bnT_ref, o_ref):
    o_ref[...] = jnp.dot(an_ref[...], bnT_ref[...],
                         preferred_element_type=jnp.float32) * (1.0 / TEMP)


def _head2(an, bnT):
    nsel, h = an.shape
    _, c = bnT.shape
    return pl.pallas_call(
        _head2_body,
        out_shape=jax.ShapeDtypeStruct((nsel, c), jnp.float32),
        interpret=_INTERPRET,
    )(an, bnT)


def kernel(x, edge_index, node_idx, labels, p_hol, p_shared, combine_weight,
           p_balance, W1, W2, alpha):
    n, d = x.shape
    kk = 33
    kp = 64
    rblk = 256
    npad = ((n + rblk - 1) // rblk) * rblk
    c = 64
    src, dst = edge_index[0], edge_index[1]
    e = src.shape[0]

    pc = combine_weight[0, 0] * p_hol + combine_weight[0, 1] * p_shared
    x_pad = jnp.pad(x, ((0, npad - n), (0, 0)))
    fea_al = _fused_prompt(x_pad, pc)

    ones_e = jnp.ones((e,), jnp.float32)
    degp = _edge_scatter(jnp.ones((npad, d), jnp.float32), src, dst, ones_e)
    deg = 1.0 + degp[0, :, 0] + degp[1, :, 0]
    dis = deg ** -0.5
    dis2 = dis * dis
    w_e = dis[src] * dis[dst]

    aggp = _edge_scatter(fea_al, src, dst, w_e)

    hn_pad = _hn_kernel(fea_al, aggp, dis2, p_balance)

    vals_p, idxs_p = _knn_topk(hn_pad, n, kk, kp, rblk)
    val = vals_p[:n, :kk]
    idx = idxs_p[:n, :kk]

    g = idx[idx.reshape(-1)].reshape(n, kk, kk)
    rev = jnp.any(g == jnp.arange(n, dtype=idx.dtype)[:, None, None], axis=-1)
    v = jax.nn.relu(jnp.where(rev, 0.5, 1.0) * val)

    aw_e = alpha * w_e
    a_self = alpha * dis2
    bv_flat = ((1.0 - alpha) * v).reshape(-1)
    idx_flat = idx.reshape(-1)
    row_rep = jnp.repeat(jnp.arange(n, dtype=jnp.int32), kk)

    esrc = jnp.concatenate([src, row_rep, idx_flat])
    edst = jnp.concatenate([dst, idx_flat, row_rep])
    ew = jnp.concatenate([aw_e, bv_flat, bv_flat])

    z1 = _matmul(fea_al, W1)
    p1 = _edge_scatter(z1, esrc, edst, ew)
    z2 = _combine_mm(p1, z1, a_self, W2)
    p2 = _edge_scatter(z2, esrc, edst, ew)
    out = _combine(p2, z2, a_self)

    sel = out[node_idx]
    onehotT = (labels[None, :] == jnp.arange(c, dtype=labels.dtype)[:, None]
               ).astype(jnp.float32)
    an, bn = _head1(onehotT, sel)
    return _head2(an, bn.T)
